# Initial kernel scaffold; baseline (speedup 1.0000x reference)
import jax
import jax.numpy as jnp
from jax.experimental import pallas as pl


def kernel(x, edge_index, W0, att_src0, att_dst0, b0, W1, att_src1, att_dst1, b1):
    raise NotImplementedError("write your pallas kernel here")


# speedup vs baseline: 17.5123x; 17.5123x over previous
import jax
import jax.numpy as jnp
from jax import lax
from jax.experimental import pallas as pl
from jax.experimental.pallas import tpu as pltpu
from jax.experimental.pallas import tpu_sc as plsc

N = 10000
D = 128
NP = 10240
NR = NP // 128
NC = 2
NS = 16
NW = NC * NS
G = 64
NB = 162
EP = NW * NB * G
BLK = 2048
GRID = NP // BLK
RB = NP // NS

f32 = jnp.float32
i32 = jnp.int32


def _attn_tail(i, h, asv_ref, adv_ref, h_ref, as_ref, ad_ref, gm_ref):
    h_ref[...] = h
    a_s = jnp.sum(h * asv_ref[...], axis=1)
    a_d = jnp.sum(h * adv_ref[...], axis=1)
    as_ref[...] = a_s.reshape(BLK // 128, 128)
    ad_ref[...] = a_d.reshape(BLK // 128, 128)
    bm = jnp.max(a_s)
    prev = jnp.where(i == 0, jnp.full((1, 128), -jnp.inf, f32), gm_ref[...])
    gm_ref[...] = jnp.maximum(prev, bm)


def _head_body(x_ref, w_ref, asv_ref, adv_ref, h_ref, as_ref, ad_ref, gm_ref):
    i = pl.program_id(0)
    h = jnp.dot(x_ref[...], w_ref[...], preferred_element_type=f32)
    _attn_tail(i, h, asv_ref, adv_ref, h_ref, as_ref, ad_ref, gm_ref)


def _mid_body(p_ref, d_ref, b_ref, w_ref, asv_ref, adv_ref,
              h_ref, as_ref, ad_ref, gm_ref):
    i = pl.program_id(0)
    num = p_ref[0] + p_ref[1]
    den = d_ref[0] + d_ref[1]
    hprev = jnp.maximum(num / (den + 1e-16) + b_ref[...], 0.0)
    h = jnp.dot(hprev, w_ref[...], preferred_element_type=f32)
    _attn_tail(i, h, asv_ref, adv_ref, h_ref, as_ref, ad_ref, gm_ref)


def _fin_body(p_ref, d_ref, b_ref, o_ref):
    num = p_ref[0] + p_ref[1]
    den = d_ref[0] + d_ref[1]
    o_ref[...] = num / (den + 1e-16) + b_ref[...]


_HEAD_OUT_SPECS = [
    pl.BlockSpec((BLK, D), lambda i: (i, 0)),
    pl.BlockSpec((BLK // 128, 128), lambda i: (i, 0)),
    pl.BlockSpec((BLK // 128, 128), lambda i: (i, 0)),
    pl.BlockSpec((1, 128), lambda i: (0, 0)),
]
_HEAD_OUT_SHAPE = [
    jax.ShapeDtypeStruct((NP, D), f32),
    jax.ShapeDtypeStruct((NP // 128, 128), f32),
    jax.ShapeDtypeStruct((NP // 128, 128), f32),
    jax.ShapeDtypeStruct((1, 128), f32),
]


def _run_head(xh, W, a_s, a_d):
    return pl.pallas_call(
        _head_body,
        grid=(GRID,),
        in_specs=[
            pl.BlockSpec((BLK, D), lambda i: (i, 0)),
            pl.BlockSpec((D, D), lambda i: (0, 0)),
            pl.BlockSpec((1, D), lambda i: (0, 0)),
            pl.BlockSpec((1, D), lambda i: (0, 0)),
        ],
        out_specs=_HEAD_OUT_SPECS,
        out_shape=_HEAD_OUT_SHAPE,
    )(xh, W, a_s, a_d)


def _run_mid(p, den, b, W, a_s, a_d):
    return pl.pallas_call(
        _mid_body,
        grid=(GRID,),
        in_specs=[
            pl.BlockSpec((NC, BLK, D), lambda i: (0, i, 0)),
            pl.BlockSpec((NC, BLK, 1), lambda i: (0, i, 0)),
            pl.BlockSpec((1, D), lambda i: (0, 0)),
            pl.BlockSpec((D, D), lambda i: (0, 0)),
            pl.BlockSpec((1, D), lambda i: (0, 0)),
            pl.BlockSpec((1, D), lambda i: (0, 0)),
        ],
        out_specs=_HEAD_OUT_SPECS,
        out_shape=_HEAD_OUT_SHAPE,
    )(p, den, b, W, a_s, a_d)


def _run_fin(p, den, b):
    return pl.pallas_call(
        _fin_body,
        grid=(GRID,),
        in_specs=[
            pl.BlockSpec((NC, BLK, D), lambda i: (0, i, 0)),
            pl.BlockSpec((NC, BLK, 1), lambda i: (0, i, 0)),
            pl.BlockSpec((1, D), lambda i: (0, 0)),
        ],
        out_specs=pl.BlockSpec((BLK, D), lambda i: (i, 0)),
        out_shape=jax.ShapeDtypeStruct((NP, D), f32),
    )(p, den, b)


def _sc_body(h_hbm, as_hbm, ad_hbm, gm_hbm, src_hbm, dst_hbm,
             out_hbm, den_hbm,
             asv, adv, gbuf, sidx, didx, wbuf, rb, denv, irows, accum, dacc):
    c = lax.axis_index("c")
    s = lax.axis_index("s")
    wid = s * NC + c
    zv = jnp.zeros((16,), f32)

    def _zrow(j, carry):
        for k in range(D // 16):
            rb[j, pl.ds(16 * k, 16)] = zv
        return carry

    lax.fori_loop(0, G, _zrow, 0)

    def _zden(j, carry):
        for k in range(D // 16):
            denv[j, pl.ds(16 * k, 16)] = zv
        return carry

    lax.fori_loop(0, NR, _zden, 0)
    iota16 = lax.iota(i32, 16)
    for t in range(NR // 16):
        irows[pl.ds(16 * t, 16)] = iota16 + 16 * t
    for t in range(RB // G):
        pltpu.sync_copy(rb, accum.at[pl.ds(s * RB + t * G, G)])

    @pl.when(s == 0)
    def _():
        pltpu.sync_copy(denv, dacc)

    plsc.subcore_barrier()

    pltpu.sync_copy(as_hbm, asv)
    pltpu.sync_copy(ad_hbm, adv)
    pltpu.sync_copy(gm_hbm, gbuf)
    gvec = gbuf[...]

    def _batch(b, carry):
        pltpu.sync_copy(src_hbm.at[wid, b], sidx)
        pltpu.sync_copy(dst_hbm.at[wid, b], didx)
        pltpu.sync_copy(h_hbm.at[sidx], rb)
        for g in range(G // 16):
            s16 = sidx[pl.ds(16 * g, 16)]
            d16 = didx[pl.ds(16 * g, 16)]
            a_s = plsc.load_gather(asv, [s16])
            a_d = plsc.load_gather(adv, [d16])
            z = a_s + a_d
            e = jnp.maximum(z, 0.2 * z)
            mz = a_d + gvec
            m = jnp.maximum(mz, 0.2 * mz)
            w = jnp.exp(e - m)
            wbuf[pl.ds(16 * g, 16)] = w
            plsc.addupdate_scatter(
                denv, [lax.shift_right_logical(d16, 7),
                       lax.bitwise_and(d16, 127)], w)

        def _srow(j, cc):
            wj = plsc.load_gather(wbuf, [jnp.full((16,), j, i32)])
            for k in range(D // 16):
                rb[j, pl.ds(16 * k, 16)] = rb[j, pl.ds(16 * k, 16)] * wj
            return cc

        lax.fori_loop(0, G, _srow, 0)
        pltpu.sync_copy(rb, accum.at[didx], add=True)
        return carry

    lax.fori_loop(0, NB, _batch, 0)

    plsc.subcore_barrier()
    pltpu.sync_copy(denv, dacc.at[irows], add=True)
    plsc.subcore_barrier()

    for t in range(RB // G):
        pltpu.sync_copy(accum.at[pl.ds(s * RB + t * G, G)],
                        out_hbm.at[c, pl.ds(s * RB + t * G, G)])

    @pl.when(s == 0)
    def _():
        pltpu.sync_copy(dacc, den_hbm.at[c])


def _run_sc(h, asf, adf, gm16, src_m, dst_m):
    kfn = pl.kernel(
        _sc_body,
        out_type=[
            jax.ShapeDtypeStruct((NC, NP, D), f32),
            jax.ShapeDtypeStruct((NC, NR, 128), f32),
        ],
        mesh=plsc.VectorSubcoreMesh(core_axis_name="c", subcore_axis_name="s",
                                    num_cores=NC, num_subcores=NS),
        compiler_params=pltpu.CompilerParams(needs_layout_passes=False),
        scratch_types=[
            pltpu.VMEM((NP,), f32),
            pltpu.VMEM((NP,), f32),
            pltpu.VMEM((16,), f32),
            pltpu.VMEM((G,), i32),
            pltpu.VMEM((G,), i32),
            pltpu.VMEM((G,), f32),
            pltpu.VMEM((G, D), f32),
            pltpu.VMEM((NR, 128), f32),
            pltpu.VMEM((NR,), i32),
            pltpu.VMEM_SHARED((NP, D), f32),
            pltpu.VMEM_SHARED((NR, 128), f32),
        ],
    )
    return kfn(h, asf, adf, gm16, src_m, dst_m)


def kernel(x, edge_index, W0, att_src0, att_dst0, b0, W1, att_src1, att_dst1, b1):
    xh = jnp.zeros((NP, D), f32).at[:N].set(x)
    loop = jnp.arange(N, dtype=i32)
    npad = EP - (edge_index.shape[1] + N)
    srcf = jnp.concatenate([edge_index[0], loop, jnp.zeros((npad,), i32)])
    dstf = jnp.concatenate([edge_index[1], loop, jnp.full((npad,), N, i32)])
    src_m = srcf.reshape(NW, NB, G)
    dst_m = dstf.reshape(NW, NB, G)

    h0, as0, ad0, gm0 = _run_head(xh, W0, att_src0.reshape(1, D),
                                  att_dst0.reshape(1, D))
    p0, den0 = _run_sc(h0, as0.reshape(NP), ad0.reshape(NP),
                       gm0.reshape(128)[:16], src_m, dst_m)
    h1, as1, ad1, gm1 = _run_mid(p0, den0.reshape(NC, NP, 1), b0.reshape(1, D),
                                 W1, att_src1.reshape(1, D), att_dst1.reshape(1, D))
    p1, den1 = _run_sc(h1, as1.reshape(NP), ad1.reshape(NP),
                       gm1.reshape(128)[:16], src_m, dst_m)
    out = _run_fin(p1, den1.reshape(NC, NP, 1), b1.reshape(1, D))
    return out[:N]


# speedup vs baseline: 26.2306x; 1.4978x over previous
import jax
import jax.numpy as jnp
from jax import lax
from jax.experimental import pallas as pl
from jax.experimental.pallas import tpu as pltpu
from jax.experimental.pallas import tpu_sc as plsc

N = 10000
D = 128
NP = 10240
NR = NP // 128
NC = 2
NS = 16
NW = NC * NS
G = 32
NB = 324
EP = NW * NB * G
BLK = 2048
GRID = NP // BLK
RB = NP // NS

f32 = jnp.float32
i32 = jnp.int32


def _attn_tail(i, h, asv_ref, adv_ref, h_ref, as_ref, ad_ref, gm_ref):
    h_ref[...] = h
    a_s = jnp.sum(h * asv_ref[...], axis=1)
    a_d = jnp.sum(h * adv_ref[...], axis=1)
    as_ref[...] = a_s.reshape(BLK // 128, 128)
    ad_ref[...] = a_d.reshape(BLK // 128, 128)
    bm = jnp.max(a_s)
    prev = jnp.where(i == 0, jnp.full((1, 128), -jnp.inf, f32), gm_ref[...])
    gm_ref[...] = jnp.maximum(prev, bm)


def _head_body(x_ref, w_ref, asv_ref, adv_ref, h_ref, as_ref, ad_ref, gm_ref):
    i = pl.program_id(0)
    h = jnp.dot(x_ref[...], w_ref[...], preferred_element_type=f32)
    _attn_tail(i, h, asv_ref, adv_ref, h_ref, as_ref, ad_ref, gm_ref)


def _mid_body(p_ref, d_ref, b_ref, w_ref, asv_ref, adv_ref,
              h_ref, as_ref, ad_ref, gm_ref):
    i = pl.program_id(0)
    num = p_ref[0] + p_ref[1]
    den = d_ref[0] + d_ref[1]
    hprev = jnp.maximum(num / (den + 1e-16) + b_ref[...], 0.0)
    h = jnp.dot(hprev, w_ref[...], preferred_element_type=f32)
    _attn_tail(i, h, asv_ref, adv_ref, h_ref, as_ref, ad_ref, gm_ref)


def _fin_body(p_ref, d_ref, b_ref, o_ref):
    num = p_ref[0] + p_ref[1]
    den = d_ref[0] + d_ref[1]
    o_ref[...] = num / (den + 1e-16) + b_ref[...]


_HEAD_OUT_SPECS = [
    pl.BlockSpec((BLK, D), lambda i: (i, 0)),
    pl.BlockSpec((BLK // 128, 128), lambda i: (i, 0)),
    pl.BlockSpec((BLK // 128, 128), lambda i: (i, 0)),
    pl.BlockSpec((1, 128), lambda i: (0, 0)),
]
_HEAD_OUT_SHAPE = [
    jax.ShapeDtypeStruct((NP, D), f32),
    jax.ShapeDtypeStruct((NP // 128, 128), f32),
    jax.ShapeDtypeStruct((NP // 128, 128), f32),
    jax.ShapeDtypeStruct((1, 128), f32),
]


def _run_head(xh, W, a_s, a_d):
    return pl.pallas_call(
        _head_body,
        grid=(GRID,),
        in_specs=[
            pl.BlockSpec((BLK, D), lambda i: (i, 0)),
            pl.BlockSpec((D, D), lambda i: (0, 0)),
            pl.BlockSpec((1, D), lambda i: (0, 0)),
            pl.BlockSpec((1, D), lambda i: (0, 0)),
        ],
        out_specs=_HEAD_OUT_SPECS,
        out_shape=_HEAD_OUT_SHAPE,
    )(xh, W, a_s, a_d)


def _run_mid(p, den, b, W, a_s, a_d):
    return pl.pallas_call(
        _mid_body,
        grid=(GRID,),
        in_specs=[
            pl.BlockSpec((NC, BLK, D), lambda i: (0, i, 0)),
            pl.BlockSpec((NC, BLK, 1), lambda i: (0, i, 0)),
            pl.BlockSpec((1, D), lambda i: (0, 0)),
            pl.BlockSpec((D, D), lambda i: (0, 0)),
            pl.BlockSpec((1, D), lambda i: (0, 0)),
            pl.BlockSpec((1, D), lambda i: (0, 0)),
        ],
        out_specs=_HEAD_OUT_SPECS,
        out_shape=_HEAD_OUT_SHAPE,
    )(p, den, b, W, a_s, a_d)


def _run_fin(p, den, b):
    return pl.pallas_call(
        _fin_body,
        grid=(GRID,),
        in_specs=[
            pl.BlockSpec((NC, BLK, D), lambda i: (0, i, 0)),
            pl.BlockSpec((NC, BLK, 1), lambda i: (0, i, 0)),
            pl.BlockSpec((1, D), lambda i: (0, 0)),
        ],
        out_specs=pl.BlockSpec((BLK, D), lambda i: (i, 0)),
        out_shape=jax.ShapeDtypeStruct((NP, D), f32),
    )(p, den, b)


def _sc_body(h_hbm, as_hbm, ad_hbm, gm_hbm, e_hbm,
             out_hbm, den_hbm,
             asv, adv, gbuf, sd0, sd1, sd2, wbuf, rb0, rb1, rb2,
             denv, irows, accum, dacc,
             gs0, gs1, gs2, ss0, ss1, ss2):
    c = lax.axis_index("c")
    s = lax.axis_index("s")
    wid = s * NC + c
    zv = jnp.zeros((16,), f32)
    sd = (sd0, sd1, sd2)
    rb = (rb0, rb1, rb2)
    gsem = (gs0, gs1, gs2)
    ssem = (ss0, ss1, ss2)

    def _zrow(j, carry):
        for k in range(D // 16):
            rb0[j, pl.ds(16 * k, 16)] = zv
        return carry

    lax.fori_loop(0, G, _zrow, 0)

    def _zden(j, carry):
        for k in range(D // 16):
            denv[j, pl.ds(16 * k, 16)] = zv
        return carry

    lax.fori_loop(0, NR, _zden, 0)
    iota16 = lax.iota(i32, 16)
    for t in range(NR // 16):
        irows[pl.ds(16 * t, 16)] = iota16 + 16 * t
    for t in range(RB // G):
        pltpu.sync_copy(rb0, accum.at[pl.ds(s * RB + t * G, G)])

    @pl.when(s == 0)
    def _():
        pltpu.sync_copy(denv, dacc)

    plsc.subcore_barrier()

    pltpu.sync_copy(as_hbm, asv)
    pltpu.sync_copy(ad_hbm, adv)
    pltpu.sync_copy(gm_hbm, gbuf)
    gvec = gbuf[...]

    pltpu.sync_copy(e_hbm.at[wid, 0], sd0)
    pltpu.sync_copy(e_hbm.at[wid, 1], sd1)
    pltpu.async_copy(h_hbm.at[sd0.at[0]], rb0, gs0)
    pltpu.async_copy(h_hbm.at[sd1.at[0]], rb1, gs1)

    def _do_batch(b, ph):
        sdp, rbp = sd[ph], rb[ph]
        pltpu.make_async_copy(h_hbm.at[sdp.at[0]], rbp, gsem[ph]).wait()
        for g in range(G // 16):
            s16 = sdp[0, pl.ds(16 * g, 16)]
            d16 = sdp[1, pl.ds(16 * g, 16)]
            a_s = plsc.load_gather(asv, [s16])
            a_d = plsc.load_gather(adv, [d16])
            z = a_s + a_d
            e = jnp.maximum(z, 0.2 * z)
            mz = a_d + gvec
            m = jnp.maximum(mz, 0.2 * mz)
            w = jnp.exp(e - m)
            wbuf[pl.ds(16 * g, 16)] = w
            plsc.addupdate_scatter(
                denv, [lax.shift_right_logical(d16, 7),
                       lax.bitwise_and(d16, 127)], w)

        def _srow(j, cc):
            wj = plsc.load_gather(wbuf, [jnp.full((16,), j, i32)])
            for k in range(D // 16):
                rbp[j, pl.ds(16 * k, 16)] = rbp[j, pl.ds(16 * k, 16)] * wj
            return cc

        lax.fori_loop(0, G, _srow, 0)
        pltpu.async_copy(rbp, accum.at[sdp.at[1]], ssem[ph], add=True)

        pv = (ph + 2) % 3
        @pl.when(b >= 1)
        def _():
            pltpu.make_async_copy(rb[pv], accum.at[sd[pv].at[1]],
                                  ssem[pv]).wait()

        @pl.when(b + 2 < NB)
        def _():
            pltpu.sync_copy(e_hbm.at[wid, b + 2], sd[pv])
            pltpu.async_copy(h_hbm.at[sd[pv].at[0]], rb[pv], gsem[pv])

    def _ring(t, carry):
        for ph in range(3):
            _do_batch(3 * t + ph, ph)
        return carry

    lax.fori_loop(0, NB // 3, _ring, 0)
    last = (NB - 1) % 3
    pltpu.make_async_copy(rb[last], accum.at[sd[last].at[1]],
                          ssem[last]).wait()

    plsc.subcore_barrier()
    pltpu.sync_copy(denv, dacc.at[irows], add=True)
    plsc.subcore_barrier()

    for t in range(RB // G):
        pltpu.sync_copy(accum.at[pl.ds(s * RB + t * G, G)],
                        out_hbm.at[c, pl.ds(s * RB + t * G, G)])

    @pl.when(s == 0)
    def _():
        pltpu.sync_copy(dacc, den_hbm.at[c])


def _run_sc(h, asf, adf, gm16, e_m):
    kfn = pl.kernel(
        _sc_body,
        out_type=[
            jax.ShapeDtypeStruct((NC, NP, D), f32),
            jax.ShapeDtypeStruct((NC, NR, 128), f32),
        ],
        mesh=plsc.VectorSubcoreMesh(core_axis_name="c", subcore_axis_name="s",
                                    num_cores=NC, num_subcores=NS),
        compiler_params=pltpu.CompilerParams(needs_layout_passes=False),
        scratch_types=[
            pltpu.VMEM((NP,), f32),
            pltpu.VMEM((NP,), f32),
            pltpu.VMEM((16,), f32),
            pltpu.VMEM((2, G), i32),
            pltpu.VMEM((2, G), i32),
            pltpu.VMEM((2, G), i32),
            pltpu.VMEM((G,), f32),
            pltpu.VMEM((G, D), f32),
            pltpu.VMEM((G, D), f32),
            pltpu.VMEM((G, D), f32),
            pltpu.VMEM((NR, 128), f32),
            pltpu.VMEM((NR,), i32),
            pltpu.VMEM_SHARED((NP, D), f32),
            pltpu.VMEM_SHARED((NR, 128), f32),
            pltpu.SemaphoreType.DMA,
            pltpu.SemaphoreType.DMA,
            pltpu.SemaphoreType.DMA,
            pltpu.SemaphoreType.DMA,
            pltpu.SemaphoreType.DMA,
            pltpu.SemaphoreType.DMA,
        ],
    )
    return kfn(h, asf, adf, gm16, e_m)


def kernel(x, edge_index, W0, att_src0, att_dst0, b0, W1, att_src1, att_dst1, b1):
    xh = jnp.zeros((NP, D), f32).at[:N].set(x)
    loop = jnp.arange(N, dtype=i32)
    npad = EP - (edge_index.shape[1] + N)
    srcf = jnp.concatenate([edge_index[0], loop, jnp.zeros((npad,), i32)])
    dstf = jnp.concatenate([edge_index[1], loop, jnp.full((npad,), N, i32)])
    e_m = jnp.stack([srcf.reshape(NW, NB, G), dstf.reshape(NW, NB, G)], axis=2)

    h0, as0, ad0, gm0 = _run_head(xh, W0, att_src0.reshape(1, D),
                                  att_dst0.reshape(1, D))
    p0, den0 = _run_sc(h0, as0.reshape(NP), ad0.reshape(NP),
                       gm0.reshape(128)[:16], e_m)
    h1, as1, ad1, gm1 = _run_mid(p0, den0.reshape(NC, NP, 1), b0.reshape(1, D),
                                 W1, att_src1.reshape(1, D), att_dst1.reshape(1, D))
    p1, den1 = _run_sc(h1, as1.reshape(NP), ad1.reshape(NP),
                       gm1.reshape(128)[:16], e_m)
    out = _run_fin(p1, den1.reshape(NC, NP, 1), b1.reshape(1, D))
    return out[:N]


# speedup vs baseline: 28.3032x; 1.0790x over previous
import jax
import jax.numpy as jnp
from jax import lax
from jax.experimental import pallas as pl
from jax.experimental.pallas import tpu as pltpu
from jax.experimental.pallas import tpu_sc as plsc

N = 10000
D = 128
NP = 10240
NR = NP // 128
NC = 2
NS = 16
NW = NC * NS
G = 32
NB = 324
EP = NW * NB * G
BLK = 2048
GRID = NP // BLK
RB = NP // NS

f32 = jnp.float32
i32 = jnp.int32


def _attn_tail(i, h, asv_ref, adv_ref, h_ref, as_ref, ad_ref, gm_ref):
    h_ref[...] = h
    a_s = jnp.sum(h * asv_ref[...], axis=1)
    a_d = jnp.sum(h * adv_ref[...], axis=1)
    as_ref[...] = a_s.reshape(BLK // 128, 128)
    ad_ref[...] = a_d.reshape(BLK // 128, 128)
    bm = jnp.max(a_s)
    prev = jnp.where(i == 0, jnp.full((1, 128), -jnp.inf, f32), gm_ref[...])
    gm_ref[...] = jnp.maximum(prev, bm)


def _head_body(x_ref, w_ref, asv_ref, adv_ref, h_ref, as_ref, ad_ref, gm_ref):
    i = pl.program_id(0)
    h = jnp.dot(x_ref[...], w_ref[...], preferred_element_type=f32)
    _attn_tail(i, h, asv_ref, adv_ref, h_ref, as_ref, ad_ref, gm_ref)


def _mid_body(p_ref, d_ref, b_ref, w_ref, asv_ref, adv_ref,
              h_ref, as_ref, ad_ref, gm_ref):
    i = pl.program_id(0)
    num = p_ref[0] + p_ref[1]
    den = d_ref[0] + d_ref[1]
    hprev = jnp.maximum(num / (den + 1e-16) + b_ref[...], 0.0)
    h = jnp.dot(hprev, w_ref[...], preferred_element_type=f32)
    _attn_tail(i, h, asv_ref, adv_ref, h_ref, as_ref, ad_ref, gm_ref)


def _fin_body(p_ref, d_ref, b_ref, o_ref):
    num = p_ref[0] + p_ref[1]
    den = d_ref[0] + d_ref[1]
    o_ref[...] = num / (den + 1e-16) + b_ref[...]


_HEAD_OUT_SPECS = [
    pl.BlockSpec((BLK, D), lambda i: (i, 0)),
    pl.BlockSpec((BLK // 128, 128), lambda i: (i, 0)),
    pl.BlockSpec((BLK // 128, 128), lambda i: (i, 0)),
    pl.BlockSpec((1, 128), lambda i: (0, 0)),
]
_HEAD_OUT_SHAPE = [
    jax.ShapeDtypeStruct((NP, D), f32),
    jax.ShapeDtypeStruct((NP // 128, 128), f32),
    jax.ShapeDtypeStruct((NP // 128, 128), f32),
    jax.ShapeDtypeStruct((1, 128), f32),
]


def _run_head(xh, W, a_s, a_d):
    return pl.pallas_call(
        _head_body,
        grid=(GRID,),
        in_specs=[
            pl.BlockSpec((BLK, D), lambda i: (i, 0)),
            pl.BlockSpec((D, D), lambda i: (0, 0)),
            pl.BlockSpec((1, D), lambda i: (0, 0)),
            pl.BlockSpec((1, D), lambda i: (0, 0)),
        ],
        out_specs=_HEAD_OUT_SPECS,
        out_shape=_HEAD_OUT_SHAPE,
    )(xh, W, a_s, a_d)


def _run_mid(p, den, b, W, a_s, a_d):
    return pl.pallas_call(
        _mid_body,
        grid=(GRID,),
        in_specs=[
            pl.BlockSpec((NC, BLK, D), lambda i: (0, i, 0)),
            pl.BlockSpec((NC, BLK, 1), lambda i: (0, i, 0)),
            pl.BlockSpec((1, D), lambda i: (0, 0)),
            pl.BlockSpec((D, D), lambda i: (0, 0)),
            pl.BlockSpec((1, D), lambda i: (0, 0)),
            pl.BlockSpec((1, D), lambda i: (0, 0)),
        ],
        out_specs=_HEAD_OUT_SPECS,
        out_shape=_HEAD_OUT_SHAPE,
    )(p, den, b, W, a_s, a_d)


def _run_fin(p, den, b):
    return pl.pallas_call(
        _fin_body,
        grid=(GRID,),
        in_specs=[
            pl.BlockSpec((NC, BLK, D), lambda i: (0, i, 0)),
            pl.BlockSpec((NC, BLK, 1), lambda i: (0, i, 0)),
            pl.BlockSpec((1, D), lambda i: (0, 0)),
        ],
        out_specs=pl.BlockSpec((BLK, D), lambda i: (i, 0)),
        out_shape=jax.ShapeDtypeStruct((NP, D), f32),
    )(p, den, b)


def _sc_body(h_hbm, as_hbm, ad_hbm, gm_hbm, e_hbm,
             out_hbm, den_hbm,
             asv, adv, gbuf, sd0, sd1, sd2, wbuf, rb0, rb1, rb2,
             denv, irows, accum, dacc,
             gs0, gs1, gs2, ss0, ss1, ss2):
    c = lax.axis_index("c")
    s = lax.axis_index("s")
    wid = s * NC + c
    zv = jnp.zeros((16,), f32)
    sd = (sd0, sd1, sd2)
    rb = (rb0, rb1, rb2)
    gsem = (gs0, gs1, gs2)
    ssem = (ss0, ss1, ss2)

    def _zrow(j, carry):
        for k in range(D // 16):
            rb0[j, pl.ds(16 * k, 16)] = zv
        return carry

    lax.fori_loop(0, G, _zrow, 0)

    def _zden(j, carry):
        for k in range(D // 16):
            denv[j, pl.ds(16 * k, 16)] = zv
        return carry

    lax.fori_loop(0, NR, _zden, 0)
    iota16 = lax.iota(i32, 16)
    for t in range(NR // 16):
        irows[pl.ds(16 * t, 16)] = iota16 + 16 * t
    for t in range(RB // G):
        pltpu.sync_copy(rb0, accum.at[pl.ds(s * RB + t * G, G)])

    @pl.when(s == 0)
    def _():
        pltpu.sync_copy(denv, dacc)

    plsc.subcore_barrier()

    pltpu.sync_copy(as_hbm, asv)
    pltpu.sync_copy(ad_hbm, adv)
    pltpu.sync_copy(gm_hbm, gbuf)
    gvec = gbuf[...]

    pltpu.sync_copy(e_hbm.at[wid, 0], sd0)
    pltpu.sync_copy(e_hbm.at[wid, 1], sd1)
    pltpu.async_copy(h_hbm.at[sd0.at[0]], rb0, gs0)
    pltpu.async_copy(h_hbm.at[sd1.at[0]], rb1, gs1)

    def _do_batch(b, ph):
        sdp, rbp = sd[ph], rb[ph]
        pltpu.make_async_copy(h_hbm.at[sdp.at[0]], rbp, gsem[ph]).wait()
        for g in range(G // 16):
            s16 = sdp[0, pl.ds(16 * g, 16)]
            d16 = sdp[1, pl.ds(16 * g, 16)]
            a_s = plsc.load_gather(asv, [s16])
            a_d = plsc.load_gather(adv, [d16])
            z = a_s + a_d
            e = jnp.maximum(z, 0.2 * z)
            mz = a_d + gvec
            m = jnp.maximum(mz, 0.2 * mz)
            w = jnp.exp(e - m)
            wbuf[pl.ds(16 * g, 16)] = w
            plsc.addupdate_scatter(
                denv, [lax.shift_right_logical(d16, 7),
                       lax.bitwise_and(d16, 127)], w)

        @plsc.parallel_loop(0, G, step=1, unroll=4)
        def _srow(j):
            wj = plsc.load_gather(wbuf, [jnp.full((16,), j, i32)])
            for k in range(D // 16):
                rbp[j, pl.ds(16 * k, 16)] = rbp[j, pl.ds(16 * k, 16)] * wj
        pltpu.async_copy(rbp, accum.at[sdp.at[1]], ssem[ph], add=True)

        pv = (ph + 2) % 3
        @pl.when(b >= 1)
        def _():
            pltpu.make_async_copy(rb[pv], accum.at[sd[pv].at[1]],
                                  ssem[pv]).wait()

        @pl.when(b + 2 < NB)
        def _():
            pltpu.sync_copy(e_hbm.at[wid, b + 2], sd[pv])
            pltpu.async_copy(h_hbm.at[sd[pv].at[0]], rb[pv], gsem[pv])

    def _ring(t, carry):
        for ph in range(3):
            _do_batch(3 * t + ph, ph)
        return carry

    lax.fori_loop(0, NB // 3, _ring, 0)
    last = (NB - 1) % 3
    pltpu.make_async_copy(rb[last], accum.at[sd[last].at[1]],
                          ssem[last]).wait()

    plsc.subcore_barrier()
    pltpu.sync_copy(denv, dacc.at[irows], add=True)
    plsc.subcore_barrier()

    for t in range(RB // G):
        pltpu.sync_copy(accum.at[pl.ds(s * RB + t * G, G)],
                        out_hbm.at[c, pl.ds(s * RB + t * G, G)])

    @pl.when(s == 0)
    def _():
        pltpu.sync_copy(dacc, den_hbm.at[c])


def _run_sc(h, asf, adf, gm16, e_m):
    kfn = pl.kernel(
        _sc_body,
        out_type=[
            jax.ShapeDtypeStruct((NC, NP, D), f32),
            jax.ShapeDtypeStruct((NC, NR, 128), f32),
        ],
        mesh=plsc.VectorSubcoreMesh(core_axis_name="c", subcore_axis_name="s",
                                    num_cores=NC, num_subcores=NS),
        compiler_params=pltpu.CompilerParams(needs_layout_passes=False),
        scratch_types=[
            pltpu.VMEM((NP,), f32),
            pltpu.VMEM((NP,), f32),
            pltpu.VMEM((16,), f32),
            pltpu.VMEM((2, G), i32),
            pltpu.VMEM((2, G), i32),
            pltpu.VMEM((2, G), i32),
            pltpu.VMEM((G,), f32),
            pltpu.VMEM((G, D), f32),
            pltpu.VMEM((G, D), f32),
            pltpu.VMEM((G, D), f32),
            pltpu.VMEM((NR, 128), f32),
            pltpu.VMEM((NR,), i32),
            pltpu.VMEM_SHARED((NP, D), f32),
            pltpu.VMEM_SHARED((NR, 128), f32),
            pltpu.SemaphoreType.DMA,
            pltpu.SemaphoreType.DMA,
            pltpu.SemaphoreType.DMA,
            pltpu.SemaphoreType.DMA,
            pltpu.SemaphoreType.DMA,
            pltpu.SemaphoreType.DMA,
        ],
    )
    return kfn(h, asf, adf, gm16, e_m)


def kernel(x, edge_index, W0, att_src0, att_dst0, b0, W1, att_src1, att_dst1, b1):
    xh = jnp.zeros((NP, D), f32).at[:N].set(x)
    loop = jnp.arange(N, dtype=i32)
    npad = EP - (edge_index.shape[1] + N)
    srcf = jnp.concatenate([edge_index[0], loop, jnp.zeros((npad,), i32)])
    dstf = jnp.concatenate([edge_index[1], loop, jnp.full((npad,), N, i32)])
    e_m = jnp.stack([srcf.reshape(NW, NB, G), dstf.reshape(NW, NB, G)], axis=2)

    h0, as0, ad0, gm0 = _run_head(xh, W0, att_src0.reshape(1, D),
                                  att_dst0.reshape(1, D))
    p0, den0 = _run_sc(h0, as0.reshape(NP), ad0.reshape(NP),
                       gm0.reshape(128)[:16], e_m)
    h1, as1, ad1, gm1 = _run_mid(p0, den0.reshape(NC, NP, 1), b0.reshape(1, D),
                                 W1, att_src1.reshape(1, D), att_dst1.reshape(1, D))
    p1, den1 = _run_sc(h1, as1.reshape(NP), ad1.reshape(NP),
                       gm1.reshape(128)[:16], e_m)
    out = _run_fin(p1, den1.reshape(NC, NP, 1), b1.reshape(1, D))
    return out[:N]


# speedup vs baseline: 33.6119x; 1.1876x over previous
import jax
import jax.numpy as jnp
from jax import lax
from jax.experimental import pallas as pl
from jax.experimental.pallas import tpu as pltpu
from jax.experimental.pallas import tpu_sc as plsc

N = 10000
D = 128
NP = 10240
NR = NP // 128
NC = 2
NS = 16
NW = NC * NS
G = 32
NB = 324
EP = NW * NB * G
BLK = 2048
GRID = NP // BLK
RB = NP // NS

f32 = jnp.float32
i32 = jnp.int32


def _attn_tail(i, h, asv_ref, adv_ref, h_ref, as_ref, ad_ref, gm_ref):
    h_ref[...] = h
    a_s = jnp.sum(h * asv_ref[...], axis=1)
    a_d = jnp.sum(h * adv_ref[...], axis=1)
    as_ref[...] = a_s.reshape(BLK // 128, 128)
    ad_ref[...] = a_d.reshape(BLK // 128, 128)
    bm = jnp.max(a_s)
    prev = jnp.where(i == 0, jnp.full((1, 128), -jnp.inf, f32), gm_ref[...])
    gm_ref[...] = jnp.maximum(prev, bm)


def _head_body(x_ref, w_ref, asv_ref, adv_ref, h_ref, as_ref, ad_ref, gm_ref):
    i = pl.program_id(0)
    h = jnp.dot(x_ref[...], w_ref[...], preferred_element_type=f32)
    _attn_tail(i, h, asv_ref, adv_ref, h_ref, as_ref, ad_ref, gm_ref)


def _mid_body(p_ref, d_ref, b_ref, w_ref, asv_ref, adv_ref,
              h_ref, as_ref, ad_ref, gm_ref):
    i = pl.program_id(0)
    num = p_ref[0] + p_ref[1]
    den = d_ref[0] + d_ref[1]
    hprev = jnp.maximum(num / (den + 1e-16) + b_ref[...], 0.0)
    h = jnp.dot(hprev, w_ref[...], preferred_element_type=f32)
    _attn_tail(i, h, asv_ref, adv_ref, h_ref, as_ref, ad_ref, gm_ref)


def _fin_body(p_ref, d_ref, b_ref, o_ref):
    num = p_ref[0] + p_ref[1]
    den = d_ref[0] + d_ref[1]
    o_ref[...] = num / (den + 1e-16) + b_ref[...]


_HEAD_OUT_SPECS = [
    pl.BlockSpec((BLK, D), lambda i: (i, 0)),
    pl.BlockSpec((BLK // 128, 128), lambda i: (i, 0)),
    pl.BlockSpec((BLK // 128, 128), lambda i: (i, 0)),
    pl.BlockSpec((1, 128), lambda i: (0, 0)),
]
_HEAD_OUT_SHAPE = [
    jax.ShapeDtypeStruct((NP, D), f32),
    jax.ShapeDtypeStruct((NP // 128, 128), f32),
    jax.ShapeDtypeStruct((NP // 128, 128), f32),
    jax.ShapeDtypeStruct((1, 128), f32),
]


def _run_head(xh, W, a_s, a_d):
    return pl.pallas_call(
        _head_body,
        grid=(GRID,),
        in_specs=[
            pl.BlockSpec((BLK, D), lambda i: (i, 0)),
            pl.BlockSpec((D, D), lambda i: (0, 0)),
            pl.BlockSpec((1, D), lambda i: (0, 0)),
            pl.BlockSpec((1, D), lambda i: (0, 0)),
        ],
        out_specs=_HEAD_OUT_SPECS,
        out_shape=_HEAD_OUT_SHAPE,
    )(xh, W, a_s, a_d)


def _run_mid(p, den, b, W, a_s, a_d):
    return pl.pallas_call(
        _mid_body,
        grid=(GRID,),
        in_specs=[
            pl.BlockSpec((NC, BLK, D), lambda i: (0, i, 0)),
            pl.BlockSpec((NC, BLK, 1), lambda i: (0, i, 0)),
            pl.BlockSpec((1, D), lambda i: (0, 0)),
            pl.BlockSpec((D, D), lambda i: (0, 0)),
            pl.BlockSpec((1, D), lambda i: (0, 0)),
            pl.BlockSpec((1, D), lambda i: (0, 0)),
        ],
        out_specs=_HEAD_OUT_SPECS,
        out_shape=_HEAD_OUT_SHAPE,
    )(p, den, b, W, a_s, a_d)


def _run_fin(p, den, b):
    return pl.pallas_call(
        _fin_body,
        grid=(GRID,),
        in_specs=[
            pl.BlockSpec((NC, BLK, D), lambda i: (0, i, 0)),
            pl.BlockSpec((NC, BLK, 1), lambda i: (0, i, 0)),
            pl.BlockSpec((1, D), lambda i: (0, 0)),
        ],
        out_specs=pl.BlockSpec((BLK, D), lambda i: (i, 0)),
        out_shape=jax.ShapeDtypeStruct((NP, D), f32),
    )(p, den, b)


def _sc_body(h_hbm, as_hbm, ad_hbm, gm_hbm, e_hbm,
             out_hbm, den_hbm,
             asv, adv, gbuf, sd0, sd1, sd2, sd3, sd4, sd5, wbuf,
             rb0, rb1, rb2,
             denv, irows, accum, dacc,
             gs0, gs1, gs2, ss0, ss1, ss2,
             is0, is1, is2, is3, is4, is5):
    c = lax.axis_index("c")
    s = lax.axis_index("s")
    wid = s * NC + c
    zv = jnp.zeros((16,), f32)
    sd = (sd0, sd1, sd2, sd3, sd4, sd5)
    rb = (rb0, rb1, rb2)
    gsem = (gs0, gs1, gs2)
    ssem = (ss0, ss1, ss2)
    isem = (is0, is1, is2, is3, is4, is5)

    def _zrow(j, carry):
        for k in range(D // 16):
            rb0[j, pl.ds(16 * k, 16)] = zv
        return carry

    lax.fori_loop(0, G, _zrow, 0)

    def _zden(j, carry):
        for k in range(D // 16):
            denv[j, pl.ds(16 * k, 16)] = zv
        return carry

    lax.fori_loop(0, NR, _zden, 0)
    iota16 = lax.iota(i32, 16)
    for t in range(NR // 16):
        irows[pl.ds(16 * t, 16)] = iota16 + 16 * t
    for t in range(RB // G):
        pltpu.sync_copy(rb0, accum.at[pl.ds(s * RB + t * G, G)])

    @pl.when(s == 0)
    def _():
        pltpu.sync_copy(denv, dacc)

    plsc.subcore_barrier()

    pltpu.sync_copy(as_hbm, asv)
    pltpu.sync_copy(ad_hbm, adv)
    pltpu.sync_copy(gm_hbm, gbuf)
    gvec = gbuf[...]

    for p in range(4):
        pltpu.async_copy(e_hbm.at[wid, p], sd[p], isem[p])
    for p in range(2):
        pltpu.make_async_copy(e_hbm.at[wid, p], sd[p], isem[p]).wait()
        pltpu.async_copy(h_hbm.at[sd[p].at[0]], rb[p], gsem[p])

    def _do_batch(b, ph):
        p3 = ph % 3
        sdp, rbp = sd[ph], rb[p3]
        pltpu.make_async_copy(h_hbm.at[sdp.at[0]], rbp, gsem[p3]).wait()
        for g in range(G // 16):
            s16 = sdp[0, pl.ds(16 * g, 16)]
            d16 = sdp[1, pl.ds(16 * g, 16)]
            a_s = plsc.load_gather(asv, [s16])
            a_d = plsc.load_gather(adv, [d16])
            z = a_s + a_d
            e = jnp.maximum(z, 0.2 * z)
            mz = a_d + gvec
            m = jnp.maximum(mz, 0.2 * mz)
            w = jnp.exp(e - m)
            wbuf[pl.ds(16 * g, 16)] = w
            plsc.addupdate_scatter(
                denv, [lax.shift_right_logical(d16, 7),
                       lax.bitwise_and(d16, 127)], w)

        @plsc.parallel_loop(0, G, step=1, unroll=4)
        def _srow(j):
            wj = plsc.load_gather(wbuf, [jnp.full((16,), j, i32)])
            for k in range(D // 16):
                rbp[j, pl.ds(16 * k, 16)] = rbp[j, pl.ds(16 * k, 16)] * wj
        pltpu.async_copy(rbp, accum.at[sdp.at[1]], ssem[p3], add=True)

        pv3 = (ph + 2) % 3
        @pl.when(b >= 1)
        def _():
            pltpu.make_async_copy(rb[pv3], accum.at[sd[(ph + 5) % 6].at[1]],
                                  ssem[pv3]).wait()

        @pl.when(b + 4 < NB)
        def _():
            pltpu.async_copy(e_hbm.at[wid, b + 4], sd[(ph + 4) % 6],
                             isem[(ph + 4) % 6])

        @pl.when(b + 2 < NB)
        def _():
            pltpu.make_async_copy(e_hbm.at[wid, b + 2], sd[(ph + 2) % 6],
                                  isem[(ph + 2) % 6]).wait()
            pltpu.async_copy(h_hbm.at[sd[(ph + 2) % 6].at[0]], rb[pv3],
                             gsem[pv3])

    def _ring(t, carry):
        for ph in range(6):
            _do_batch(6 * t + ph, ph)
        return carry

    lax.fori_loop(0, NB // 6, _ring, 0)
    pltpu.make_async_copy(rb[(NB - 1) % 3], accum.at[sd[(NB - 1) % 6].at[1]],
                          ssem[(NB - 1) % 3]).wait()

    plsc.subcore_barrier()
    pltpu.sync_copy(denv, dacc.at[irows], add=True)
    plsc.subcore_barrier()

    for t in range(RB // G):
        pltpu.sync_copy(accum.at[pl.ds(s * RB + t * G, G)],
                        out_hbm.at[c, pl.ds(s * RB + t * G, G)])

    @pl.when(s == 0)
    def _():
        pltpu.sync_copy(dacc, den_hbm.at[c])


def _run_sc(h, asf, adf, gm16, e_m):
    kfn = pl.kernel(
        _sc_body,
        out_type=[
            jax.ShapeDtypeStruct((NC, NP, D), f32),
            jax.ShapeDtypeStruct((NC, NR, 128), f32),
        ],
        mesh=plsc.VectorSubcoreMesh(core_axis_name="c", subcore_axis_name="s",
                                    num_cores=NC, num_subcores=NS),
        compiler_params=pltpu.CompilerParams(needs_layout_passes=False),
        scratch_types=[
            pltpu.VMEM((NP,), f32),
            pltpu.VMEM((NP,), f32),
            pltpu.VMEM((16,), f32),
            pltpu.VMEM((2, G), i32),
            pltpu.VMEM((2, G), i32),
            pltpu.VMEM((2, G), i32),
            pltpu.VMEM((2, G), i32),
            pltpu.VMEM((2, G), i32),
            pltpu.VMEM((2, G), i32),
            pltpu.VMEM((G,), f32),
            pltpu.VMEM((G, D), f32),
            pltpu.VMEM((G, D), f32),
            pltpu.VMEM((G, D), f32),
            pltpu.VMEM((NR, 128), f32),
            pltpu.VMEM((NR,), i32),
            pltpu.VMEM_SHARED((NP, D), f32),
            pltpu.VMEM_SHARED((NR, 128), f32),
            pltpu.SemaphoreType.DMA,
            pltpu.SemaphoreType.DMA,
            pltpu.SemaphoreType.DMA,
            pltpu.SemaphoreType.DMA,
            pltpu.SemaphoreType.DMA,
            pltpu.SemaphoreType.DMA,
            pltpu.SemaphoreType.DMA,
            pltpu.SemaphoreType.DMA,
            pltpu.SemaphoreType.DMA,
            pltpu.SemaphoreType.DMA,
            pltpu.SemaphoreType.DMA,
            pltpu.SemaphoreType.DMA,
        ],
    )
    return kfn(h, asf, adf, gm16, e_m)


def kernel(x, edge_index, W0, att_src0, att_dst0, b0, W1, att_src1, att_dst1, b1):
    xh = jnp.zeros((NP, D), f32).at[:N].set(x)
    loop = jnp.arange(N, dtype=i32)
    npad = EP - (edge_index.shape[1] + N)
    srcf = jnp.concatenate([edge_index[0], loop, jnp.zeros((npad,), i32)])
    dstf = jnp.concatenate([edge_index[1], loop, jnp.full((npad,), N, i32)])
    e_m = jnp.stack([srcf.reshape(NW, NB, G), dstf.reshape(NW, NB, G)], axis=2)

    h0, as0, ad0, gm0 = _run_head(xh, W0, att_src0.reshape(1, D),
                                  att_dst0.reshape(1, D))
    p0, den0 = _run_sc(h0, as0.reshape(NP), ad0.reshape(NP),
                       gm0.reshape(128)[:16], e_m)
    h1, as1, ad1, gm1 = _run_mid(p0, den0.reshape(NC, NP, 1), b0.reshape(1, D),
                                 W1, att_src1.reshape(1, D), att_dst1.reshape(1, D))
    p1, den1 = _run_sc(h1, as1.reshape(NP), ad1.reshape(NP),
                       gm1.reshape(128)[:16], e_m)
    out = _run_fin(p1, den1.reshape(NC, NP, 1), b1.reshape(1, D))
    return out[:N]
